# Initial kernel scaffold; baseline (speedup 1.0000x reference)
import jax
import jax.numpy as jnp
from jax.experimental import pallas as pl


def kernel(class_log_probs, last_log_probs, last_predictions):
    raise NotImplementedError("write your pallas kernel here")


# speedup vs baseline: 1.2696x; 1.2696x over previous
import functools

import jax
import jax.numpy as jnp
from jax import lax
from jax.experimental import pallas as pl
from jax.experimental.pallas import tpu as pltpu
from jax.experimental.pallas import tpu_sc as plsc

END = 2
BATCH = 32
BEAM = 5
VOCAB = 100000
L = 16
NC, NS = 2, 16
SEGW = 400
SEGS = VOCAB // SEGW
CHUNK = 20000
NCHUNK = VOCAB // CHUNK
SEGS_PER_CHUNK = CHUNK // SEGW
NEG_INF = float("-inf")
BIG = 2**31 - 1


def _argmax16(rv, rp):
    v = jnp.max(rv)
    p = jnp.min(jnp.where(rv == v, rp, BIG))
    return v, p


def _sc_body(clp_hbm, llp_hbm, lp_hbm,
             preds_hbm, vals_hbm, bp_hbm,
             buf0, buf1, segmax_v, union_v, llp_vm, lp_vm,
             cand_vals, cand_idx, o_preds, o_vals, o_bp,
             sem0, sem1, gsem, osem):
    w = lax.axis_index("s") * NC + lax.axis_index("c")
    lane = lax.iota(jnp.int32, L)
    minf = jnp.full((L,), NEG_INF, jnp.float32)

    pltpu.sync_copy(llp_hbm.at[w], llp_vm)
    pltpu.sync_copy(lp_hbm.at[w], lp_vm)
    llp_vec = llp_vm[...]
    lp_vec = lp_vm[...]

    bufs = (buf0, buf1)
    sems = (sem0, sem1)

    def row_body(r, _):
        rowbase = (w * BEAM + r) * VOCAB

        pltpu.make_async_copy(
            clp_hbm.at[pl.ds(rowbase, CHUNK)], bufs[0], sems[0]).start()
        for c in range(NCHUNK):
            cur, csem = bufs[c % 2], sems[c % 2]
            if c + 1 < NCHUNK:
                pltpu.make_async_copy(
                    clp_hbm.at[pl.ds(rowbase + (c + 1) * CHUNK, CHUNK)],
                    bufs[(c + 1) % 2], sems[(c + 1) % 2]).start()
            pltpu.make_async_copy(
                clp_hbm.at[pl.ds(rowbase + c * CHUNK, CHUNK)], cur, csem).wait()

            def seg_body(s, _, cur=cur, c=c):
                acc = cur[pl.ds(s * SEGW, L)]
                for j in range(1, SEGW // L):
                    acc = jnp.maximum(acc, cur[pl.ds(s * SEGW + j * L, L)])
                segmax_v[pl.ds((c * SEGS_PER_CHUNK + s) * L, L)] = acc
                return 0
            lax.fori_loop(0, SEGS_PER_CHUNK, seg_body, 0)

        found = []
        for _k in range(5):
            def p2_body(i, carry):
                rv, rp = carry
                x = segmax_v[pl.ds(i * L, L)]
                pos = i * L + lane
                m = x > rv
                return jnp.where(m, x, rv), jnp.where(m, pos, rp)
            rv, rp = lax.fori_loop(0, SEGS, p2_body,
                                   (minf, jnp.zeros((L,), jnp.int32)))
            _, p = _argmax16(rv, rp)
            parent = p >> 4
            segmax_v[pl.ds(parent * L, L)] = minf
            found.append(parent)

        for k in range(5):
            pltpu.make_async_copy(
                clp_hbm.at[pl.ds(rowbase + found[k] * SEGW, SEGW)],
                union_v.at[pl.ds(k * SEGW, SEGW)], gsem).start()
        for k in range(5):
            pltpu.make_async_copy(
                clp_hbm.at[pl.ds(rowbase + found[k] * SEGW, SEGW)],
                union_v.at[pl.ds(k * SEGW, SEGW)], gsem).wait()

        UV = 5 * SEGW // L
        cv = minf
        ci = jnp.zeros((L,), jnp.int32)
        for k in range(5):
            def p3_body(i, carry):
                rv, rp = carry
                x = union_v[pl.ds(i * L, L)]
                pos = i * L + lane
                m = x > rv
                return jnp.where(m, x, rv), jnp.where(m, pos, rp)
            rv, rp = lax.fori_loop(0, UV, p3_body,
                                   (minf, jnp.zeros((L,), jnp.int32)))
            v, p = _argmax16(rv, rp)
            plsc.store_scatter(union_v, [jnp.full((L,), p, jnp.int32)],
                               minf, mask=lane == 0)
            slot = p // SEGW
            within = p - slot * SEGW
            seg_id = jnp.int32(0)
            for k2 in range(5):
                seg_id = jnp.where(slot == k2, found[k2], seg_id)
            gid = seg_id * SEGW + within
            cv = jnp.where(lane == k, v, cv)
            ci = jnp.where(lane == k, gid, ci)

        llp_r = jnp.max(jnp.where(lane == r, llp_vec, NEG_INF))
        pred_r = jnp.max(jnp.where(lane == r, lp_vec, jnp.int32(-1)))
        ended = pred_r == END
        end_vals = jnp.where(lane == 0, jnp.float32(0.0), minf)
        end_idx = jnp.where(lane == 0, 2,
                            jnp.where(lane == 1, 0,
                                      jnp.where(lane == 2, 1, lane)))
        cv = jnp.where(ended, end_vals, cv) + llp_r
        ci = jnp.where(ended, end_idx, ci)
        cand_vals[pl.ds(r * L, L)] = cv
        cand_idx[pl.ds(r * L, L)] = ci
        return 0

    lax.fori_loop(0, BEAM, row_body, 0)

    ov = jnp.full((L,), NEG_INF, jnp.float32)
    oc = jnp.zeros((L,), jnp.int32)
    ob = jnp.zeros((L,), jnp.int32)
    for k in range(5):
        def fb(i, carry):
            rv, rp = carry
            x = cand_vals[pl.ds(i * L, L)]
            pos = i * L + lane
            m = x > rv
            return jnp.where(m, x, rv), jnp.where(m, pos, rp)
        rv, rp = lax.fori_loop(0, BEAM, fb,
                               (jnp.full((L,), NEG_INF, jnp.float32),
                                jnp.zeros((L,), jnp.int32)))
        v, p = _argmax16(rv, rp)
        plsc.store_scatter(cand_vals, [jnp.full((L,), p, jnp.int32)],
                           jnp.full((L,), NEG_INF, jnp.float32),
                           mask=lane == 0)
        cls_vec = plsc.load_gather(cand_idx, [jnp.full((L,), p, jnp.int32)])
        ov = jnp.where(lane == k, v, ov)
        oc = jnp.where(lane == k, cls_vec, oc)
        ob = jnp.where(lane == k, p >> 4, ob)

    o_preds[...] = oc
    o_vals[...] = ov
    o_bp[...] = ob
    pltpu.make_async_copy(o_preds, preds_hbm.at[w], osem).start()
    pltpu.make_async_copy(o_vals, vals_hbm.at[w], osem).start()
    pltpu.make_async_copy(o_bp, bp_hbm.at[w], osem).start()
    pltpu.make_async_copy(o_preds, preds_hbm.at[w], osem).wait()
    pltpu.make_async_copy(o_vals, vals_hbm.at[w], osem).wait()
    pltpu.make_async_copy(o_bp, bp_hbm.at[w], osem).wait()


@jax.jit
def kernel(class_log_probs, last_log_probs, last_predictions):
    clp_flat = class_log_probs.reshape(-1)
    llp_pad = jnp.pad(last_log_probs, ((0, 0), (0, L - BEAM)))
    lp_pad = jnp.pad(last_predictions.reshape(BATCH, BEAM).astype(jnp.int32),
                     ((0, 0), (0, L - BEAM)))

    mesh = plsc.VectorSubcoreMesh(core_axis_name="c", subcore_axis_name="s",
                                  num_cores=NC, num_subcores=NS)
    sc_call = pl.kernel(
        _sc_body,
        out_type=[
            jax.ShapeDtypeStruct((BATCH, L), jnp.int32),
            jax.ShapeDtypeStruct((BATCH, L), jnp.float32),
            jax.ShapeDtypeStruct((BATCH, L), jnp.int32),
        ],
        mesh=mesh,
        compiler_params=pltpu.CompilerParams(needs_layout_passes=False),
        scratch_types=[
            pltpu.VMEM((CHUNK,), jnp.float32),
            pltpu.VMEM((CHUNK,), jnp.float32),
            pltpu.VMEM((SEGS * L,), jnp.float32),
            pltpu.VMEM((5 * SEGW,), jnp.float32),
            pltpu.VMEM((L,), jnp.float32),
            pltpu.VMEM((L,), jnp.int32),
            pltpu.VMEM((BEAM * L,), jnp.float32),
            pltpu.VMEM((BEAM * L,), jnp.int32),
            pltpu.VMEM((L,), jnp.int32),
            pltpu.VMEM((L,), jnp.float32),
            pltpu.VMEM((L,), jnp.int32),
            pltpu.SemaphoreType.DMA,
            pltpu.SemaphoreType.DMA,
            pltpu.SemaphoreType.DMA,
            pltpu.SemaphoreType.DMA,
        ],
    )
    preds16, vals16, bp16 = sc_call(clp_flat, llp_pad, lp_pad)
    return (preds16[:, :BEAM], vals16[:, :BEAM], bp16[:, :BEAM])


# speedup vs baseline: 1.5944x; 1.2559x over previous
import jax
import jax.numpy as jnp
from jax import lax
from jax.experimental import pallas as pl
from jax.experimental.pallas import tpu as pltpu
from jax.experimental.pallas import tpu_sc as plsc

END = 2
BATCH = 32
BEAM = 5
VOCAB = 100000
L = 16
NC, NS = 2, 16
SEGW = 400
SEGS = VOCAB // SEGW
CHUNK = 20000
NCHUNK = VOCAB // CHUNK
SEGS_PER_CHUNK = CHUNK // SEGW
NEG_INF = float("-inf")
BIG = 2**31 - 1


def _merge5(V, P, x, pos):
    for j in range(5):
        m = x > V[j]
        nv = jnp.where(m, x, V[j])
        np_ = jnp.where(m, pos, P[j])
        x = jnp.where(m, V[j], x)
        pos = jnp.where(m, P[j], pos)
        V[j], P[j] = nv, np_
    return V, P


def _argmax5(V, P):
    rv, rp = V[0], P[0]
    for j in range(1, 5):
        m = V[j] > rv
        rv = jnp.where(m, V[j], rv)
        rp = jnp.where(m, P[j], rp)
    v = jnp.max(rv)
    p = jnp.min(jnp.where(rv == v, rp, BIG))
    return v, p


def _sc_body(clp_hbm, llp_hbm, lp_hbm,
             preds_hbm, vals_hbm, bp_hbm,
             buf0, buf1, union_v, llp_vm, lp_vm,
             cand_vals, cand_idx, o_preds, o_vals, o_bp,
             sem0, sem1, gsem, osem):
    w = lax.axis_index("s") * NC + lax.axis_index("c")
    lane = lax.iota(jnp.int32, L)
    minf = jnp.full((L,), NEG_INF, jnp.float32)
    zero_i = jnp.zeros((L,), jnp.int32)

    pltpu.sync_copy(llp_hbm.at[w], llp_vm)
    pltpu.sync_copy(lp_hbm.at[w], lp_vm)
    llp_vec = llp_vm[...]
    lp_vec = lp_vm[...]

    bufs = (buf0, buf1)
    sems = (sem0, sem1)

    def row_body(r, _):
        rowbase = (w * BEAM + r) * VOCAB

        pltpu.make_async_copy(
            clp_hbm.at[pl.ds(rowbase, CHUNK)], bufs[0], sems[0]).start()
        carry = (minf,) * 5 + (zero_i,) * 5
        for c in range(NCHUNK):
            cur, csem = bufs[c % 2], sems[c % 2]
            if c + 1 < NCHUNK:
                pltpu.make_async_copy(
                    clp_hbm.at[pl.ds(rowbase + (c + 1) * CHUNK, CHUNK)],
                    bufs[(c + 1) % 2], sems[(c + 1) % 2]).start()
            pltpu.make_async_copy(
                clp_hbm.at[pl.ds(rowbase + c * CHUNK, CHUNK)], cur, csem).wait()

            def seg_body(s, kc, cur=cur, c=c):
                V = list(kc[0:5])
                P = list(kc[5:10])
                acc = cur[pl.ds(s * SEGW, L)]
                for j in range(1, SEGW // L):
                    acc = jnp.maximum(acc, cur[pl.ds(s * SEGW + j * L, L)])
                pos = (c * SEGS_PER_CHUNK + s) * L + lane
                V, P = _merge5(V, P, acc, pos)
                return tuple(V) + tuple(P)
            carry = lax.fori_loop(0, SEGS_PER_CHUNK, seg_body, carry)

        V = list(carry[0:5])
        P = list(carry[5:10])
        found = []
        for _k in range(5):
            _, p = _argmax5(V, P)
            parent = p >> 4
            found.append(parent)
            for j in range(5):
                V[j] = jnp.where((P[j] >> 4) == parent, NEG_INF, V[j])

        for k in range(5):
            pltpu.make_async_copy(
                clp_hbm.at[pl.ds(rowbase + found[k] * SEGW, SEGW)],
                union_v.at[pl.ds(k * SEGW, SEGW)], gsem).start()
        for k in range(5):
            pltpu.make_async_copy(
                clp_hbm.at[pl.ds(rowbase + found[k] * SEGW, SEGW)],
                union_v.at[pl.ds(k * SEGW, SEGW)], gsem).wait()

        UV = 5 * SEGW // L

        def u_body(i, kc):
            V = list(kc[0:5])
            P = list(kc[5:10])
            x = union_v[pl.ds(i * L, L)]
            pos = i * L + lane
            V, P = _merge5(V, P, x, pos)
            return tuple(V) + tuple(P)
        ucarry = lax.fori_loop(0, UV, u_body, (minf,) * 5 + (zero_i,) * 5)
        V = list(ucarry[0:5])
        P = list(ucarry[5:10])

        cv = minf
        ci = zero_i
        for k in range(5):
            v, p = _argmax5(V, P)
            for j in range(5):
                V[j] = jnp.where(P[j] == p, NEG_INF, V[j])
            slot = p // SEGW
            within = p - slot * SEGW
            seg_id = jnp.int32(0)
            for k2 in range(5):
                seg_id = jnp.where(slot == k2, found[k2], seg_id)
            gid = seg_id * SEGW + within
            cv = jnp.where(lane == k, v, cv)
            ci = jnp.where(lane == k, gid, ci)

        llp_r = jnp.max(jnp.where(lane == r, llp_vec, NEG_INF))
        pred_r = jnp.max(jnp.where(lane == r, lp_vec, jnp.int32(-1)))
        ended = pred_r == END
        end_vals = jnp.where(lane == 0, jnp.float32(0.0), minf)
        end_idx = jnp.where(lane == 0, 2,
                            jnp.where(lane == 1, 0,
                                      jnp.where(lane == 2, 1, lane)))
        cv = jnp.where(ended, end_vals, cv) + llp_r
        ci = jnp.where(ended, end_idx, ci)
        cand_vals[pl.ds(r * L, L)] = cv
        cand_idx[pl.ds(r * L, L)] = ci
        return 0

    lax.fori_loop(0, BEAM, row_body, 0)

    V = [jnp.full((L,), NEG_INF, jnp.float32) for _ in range(5)]
    P = [jnp.zeros((L,), jnp.int32) for _ in range(5)]
    for r in range(BEAM):
        x = cand_vals[pl.ds(r * L, L)]
        pos = r * L + lane
        V, P = _merge5(V, P, x, pos)

    ov = jnp.full((L,), NEG_INF, jnp.float32)
    oc = jnp.zeros((L,), jnp.int32)
    ob = jnp.zeros((L,), jnp.int32)
    for k in range(5):
        v, p = _argmax5(V, P)
        for j in range(5):
            V[j] = jnp.where(P[j] == p, NEG_INF, V[j])
        cls_vec = plsc.load_gather(cand_idx, [jnp.full((L,), p, jnp.int32)])
        ov = jnp.where(lane == k, v, ov)
        oc = jnp.where(lane == k, cls_vec, oc)
        ob = jnp.where(lane == k, p >> 4, ob)

    o_preds[...] = oc
    o_vals[...] = ov
    o_bp[...] = ob
    pltpu.make_async_copy(o_preds, preds_hbm.at[w], osem).start()
    pltpu.make_async_copy(o_vals, vals_hbm.at[w], osem).start()
    pltpu.make_async_copy(o_bp, bp_hbm.at[w], osem).start()
    pltpu.make_async_copy(o_preds, preds_hbm.at[w], osem).wait()
    pltpu.make_async_copy(o_vals, vals_hbm.at[w], osem).wait()
    pltpu.make_async_copy(o_bp, bp_hbm.at[w], osem).wait()


@jax.jit
def kernel(class_log_probs, last_log_probs, last_predictions):
    clp_flat = class_log_probs.reshape(-1)
    llp_pad = jnp.pad(last_log_probs, ((0, 0), (0, L - BEAM)))
    lp_pad = jnp.pad(last_predictions.reshape(BATCH, BEAM).astype(jnp.int32),
                     ((0, 0), (0, L - BEAM)))

    mesh = plsc.VectorSubcoreMesh(core_axis_name="c", subcore_axis_name="s",
                                  num_cores=NC, num_subcores=NS)
    sc_call = pl.kernel(
        _sc_body,
        out_type=[
            jax.ShapeDtypeStruct((BATCH, L), jnp.int32),
            jax.ShapeDtypeStruct((BATCH, L), jnp.float32),
            jax.ShapeDtypeStruct((BATCH, L), jnp.int32),
        ],
        mesh=mesh,
        compiler_params=pltpu.CompilerParams(needs_layout_passes=False),
        scratch_types=[
            pltpu.VMEM((CHUNK,), jnp.float32),
            pltpu.VMEM((CHUNK,), jnp.float32),
            pltpu.VMEM((5 * SEGW,), jnp.float32),
            pltpu.VMEM((L,), jnp.float32),
            pltpu.VMEM((L,), jnp.int32),
            pltpu.VMEM((BEAM * L,), jnp.float32),
            pltpu.VMEM((BEAM * L,), jnp.int32),
            pltpu.VMEM((L,), jnp.int32),
            pltpu.VMEM((L,), jnp.float32),
            pltpu.VMEM((L,), jnp.int32),
            pltpu.SemaphoreType.DMA,
            pltpu.SemaphoreType.DMA,
            pltpu.SemaphoreType.DMA,
            pltpu.SemaphoreType.DMA,
        ],
    )
    preds16, vals16, bp16 = sc_call(clp_flat, llp_pad, lp_pad)
    return (preds16[:, :BEAM], vals16[:, :BEAM], bp16[:, :BEAM])


# speedup vs baseline: 3.2950x; 2.0667x over previous
import jax
import jax.numpy as jnp
from jax import lax
from jax.experimental import pallas as pl
from jax.experimental.pallas import tpu as pltpu
from jax.experimental.pallas import tpu_sc as plsc

END = 2
BATCH = 32
BEAM = 5
VOCAB = 100000
L = 16
NC, NS = 2, 16
BODY = 99968
SEGW = 512
CHUNK = 19968
NCHUNK = 5
SEGS_PER_CHUNK = CHUNK // SEGW
NSEG = 196
LAST_WIN = BODY - SEGW
NEG_INF = float("-inf")
BIG = 2**31 - 1


def _merge5(V, P, x, pos):
    for j in range(5):
        m = x > V[j]
        nv = jnp.where(m, x, V[j])
        np_ = jnp.where(m, pos, P[j])
        x = jnp.where(m, V[j], x)
        pos = jnp.where(m, P[j], pos)
        V[j], P[j] = nv, np_
    return V, P


def _argmax5(V, P):
    rv, rp = V[0], P[0]
    for j in range(1, 5):
        m = V[j] > rv
        rv = jnp.where(m, V[j], rv)
        rp = jnp.where(m, P[j], rp)
    v = jnp.max(rv)
    p = jnp.min(jnp.where(rv == v, rp, BIG))
    return v, p


def _sc_body(clp_hbm, tail_hbm, llp_hbm, lp_hbm,
             preds_hbm, vals_hbm, bp_hbm,
             buf0, buf1, union_v, llp_vm, lp_vm, tail_vm, idx_vm,
             cand_vals, cand_idx, o_preds, o_vals, o_bp,
             sem0, sem1, gsem, osem):
    w = lax.axis_index("s") * NC + lax.axis_index("c")
    lane = lax.iota(jnp.int32, L)
    minf = jnp.full((L,), NEG_INF, jnp.float32)
    zero_i = jnp.zeros((L,), jnp.int32)

    pltpu.sync_copy(llp_hbm.at[w], llp_vm)
    pltpu.sync_copy(lp_hbm.at[w], lp_vm)
    llp_vec = llp_vm[...]
    lp_vec = lp_vm[...]

    bufs = (buf0, buf1)
    sems = (sem0, sem1)

    def row_body(r, _):
        g = w * BEAM + r
        idx_vm[...] = jnp.full((L,), g, jnp.int32)
        idx1 = idx_vm.at[pl.ds(0, 1)]
        pltpu.sync_copy(tail_hbm.at[g], tail_vm)

        pltpu.make_async_copy(
            clp_hbm.at[idx1, pl.ds(0, CHUNK)], bufs[0], sems[0]).start()
        carry = (minf,) * 5 + (zero_i,) * 5
        for c in range(NCHUNK):
            cur, csem = bufs[c % 2], sems[c % 2]
            if c + 1 < NCHUNK:
                pltpu.make_async_copy(
                    clp_hbm.at[idx1, pl.ds((c + 1) * CHUNK, CHUNK)],
                    bufs[(c + 1) % 2], sems[(c + 1) % 2]).start()
            else:
                pltpu.make_async_copy(
                    clp_hbm.at[idx1, pl.ds(NCHUNK * CHUNK, 128)],
                    bufs[1].at[pl.ds(0, 1), pl.ds(0, 128)], sems[1]).start()
            pltpu.make_async_copy(
                clp_hbm.at[idx1, pl.ds(c * CHUNK, CHUNK)], cur, csem).wait()

            def seg_body(s, kc, cur=cur, c=c):
                V = list(kc[0:5])
                P = list(kc[5:10])
                acc = cur[0, pl.ds(s * SEGW, L)]
                for j in range(1, SEGW // L):
                    acc = jnp.maximum(acc, cur[0, pl.ds(s * SEGW + j * L, L)])
                pos = (c * SEGS_PER_CHUNK + s) * L + lane
                V, P = _merge5(V, P, acc, pos)
                return tuple(V) + tuple(P)
            carry = lax.fori_loop(0, SEGS_PER_CHUNK, seg_body, carry)

        pltpu.make_async_copy(
            clp_hbm.at[idx1, pl.ds(NCHUNK * CHUNK, 128)],
            bufs[1].at[pl.ds(0, 1), pl.ds(0, 128)], sems[1]).wait()
        V = list(carry[0:5])
        P = list(carry[5:10])
        acc = bufs[1][0, pl.ds(0, L)]
        for j in range(1, 128 // L):
            acc = jnp.maximum(acc, bufs[1][0, pl.ds(j * L, L)])
        V, P = _merge5(V, P, acc, (NSEG - 1) * L + lane)

        found = []
        for _k in range(5):
            _, p = _argmax5(V, P)
            parent = p >> 4
            found.append(parent)
            for j in range(5):
                V[j] = jnp.where((P[j] >> 4) == parent, NEG_INF, V[j])

        offs = []
        for k in range(5):
            o = found[k] * SEGW
            o = jnp.where(o > LAST_WIN, LAST_WIN, o)
            offs.append(pl.multiple_of(o, 128))
        for k in range(5):
            pltpu.make_async_copy(
                clp_hbm.at[idx1, pl.ds(offs[k], SEGW)],
                union_v.at[pl.ds(0, 1), pl.ds(k * SEGW, SEGW)], gsem).start()
        for k in range(5):
            pltpu.make_async_copy(
                clp_hbm.at[idx1, pl.ds(offs[k], SEGW)],
                union_v.at[pl.ds(0, 1), pl.ds(k * SEGW, SEGW)], gsem).wait()

        V = [minf] * 5
        P = [zero_i] * 5
        for k in range(5):
            def win_body(i, kc, k=k, o=offs[k]):
                Vw = list(kc[0:5])
                Pw = list(kc[5:10])
                x = union_v[0, pl.ds(k * SEGW + i * L, L)]
                gid = o + i * L + lane
                Vw, Pw = _merge5(Vw, Pw, x, gid)
                return tuple(Vw) + tuple(Pw)
            kc = lax.fori_loop(0, SEGW // L, win_body, tuple(V) + tuple(P))
            V = list(kc[0:5])
            P = list(kc[5:10])
        V, P = _merge5(V, P, tail_vm[pl.ds(0, L)], BODY + lane)
        V, P = _merge5(V, P, tail_vm[pl.ds(L, L)], BODY + L + lane)

        cv = minf
        ci = zero_i
        for k in range(5):
            v, p = _argmax5(V, P)
            for j in range(5):
                V[j] = jnp.where(P[j] == p, NEG_INF, V[j])
            cv = jnp.where(lane == k, v, cv)
            ci = jnp.where(lane == k, p, ci)

        llp_r = jnp.max(jnp.where(lane == r, llp_vec, NEG_INF))
        pred_r = jnp.max(jnp.where(lane == r, lp_vec, jnp.int32(-1)))
        ended = pred_r == END
        end_vals = jnp.where(lane == 0, jnp.float32(0.0), minf)
        end_idx = jnp.where(lane == 0, 2,
                            jnp.where(lane == 1, 0,
                                      jnp.where(lane == 2, 1, lane)))
        cv = jnp.where(ended, end_vals, cv) + llp_r
        ci = jnp.where(ended, end_idx, ci)
        cand_vals[pl.ds(r * L, L)] = cv
        cand_idx[pl.ds(r * L, L)] = ci
        return 0

    lax.fori_loop(0, BEAM, row_body, 0)

    V = [jnp.full((L,), NEG_INF, jnp.float32) for _ in range(5)]
    P = [jnp.zeros((L,), jnp.int32) for _ in range(5)]
    for r in range(BEAM):
        x = cand_vals[pl.ds(r * L, L)]
        pos = r * L + lane
        V, P = _merge5(V, P, x, pos)

    ov = jnp.full((L,), NEG_INF, jnp.float32)
    oc = jnp.zeros((L,), jnp.int32)
    ob = jnp.zeros((L,), jnp.int32)
    for k in range(5):
        v, p = _argmax5(V, P)
        for j in range(5):
            V[j] = jnp.where(P[j] == p, NEG_INF, V[j])
        cls_vec = plsc.load_gather(cand_idx, [jnp.full((L,), p, jnp.int32)])
        ov = jnp.where(lane == k, v, ov)
        oc = jnp.where(lane == k, cls_vec, oc)
        ob = jnp.where(lane == k, p >> 4, ob)

    o_preds[...] = oc
    o_vals[...] = ov
    o_bp[...] = ob
    pltpu.make_async_copy(o_preds, preds_hbm.at[w], osem).start()
    pltpu.make_async_copy(o_vals, vals_hbm.at[w], osem).start()
    pltpu.make_async_copy(o_bp, bp_hbm.at[w], osem).start()
    pltpu.make_async_copy(o_preds, preds_hbm.at[w], osem).wait()
    pltpu.make_async_copy(o_vals, vals_hbm.at[w], osem).wait()
    pltpu.make_async_copy(o_bp, bp_hbm.at[w], osem).wait()


@jax.jit
def kernel(class_log_probs, last_log_probs, last_predictions):
    clp_tail = class_log_probs[:, BODY:]
    llp_pad = jnp.pad(last_log_probs, ((0, 0), (0, L - BEAM)))
    lp_pad = jnp.pad(last_predictions.reshape(BATCH, BEAM).astype(jnp.int32),
                     ((0, 0), (0, L - BEAM)))

    mesh = plsc.VectorSubcoreMesh(core_axis_name="c", subcore_axis_name="s",
                                  num_cores=NC, num_subcores=NS)
    sc_call = pl.kernel(
        _sc_body,
        out_type=[
            jax.ShapeDtypeStruct((BATCH, L), jnp.int32),
            jax.ShapeDtypeStruct((BATCH, L), jnp.float32),
            jax.ShapeDtypeStruct((BATCH, L), jnp.int32),
        ],
        mesh=mesh,
        compiler_params=pltpu.CompilerParams(needs_layout_passes=False),
        scratch_types=[
            pltpu.VMEM((1, CHUNK), jnp.float32),
            pltpu.VMEM((1, CHUNK), jnp.float32),
            pltpu.VMEM((1, 5 * SEGW), jnp.float32),
            pltpu.VMEM((L,), jnp.float32),
            pltpu.VMEM((L,), jnp.int32),
            pltpu.VMEM((VOCAB - BODY,), jnp.float32),
            pltpu.VMEM((L,), jnp.int32),
            pltpu.VMEM((BEAM * L,), jnp.float32),
            pltpu.VMEM((BEAM * L,), jnp.int32),
            pltpu.VMEM((L,), jnp.int32),
            pltpu.VMEM((L,), jnp.float32),
            pltpu.VMEM((L,), jnp.int32),
            pltpu.SemaphoreType.DMA,
            pltpu.SemaphoreType.DMA,
            pltpu.SemaphoreType.DMA,
            pltpu.SemaphoreType.DMA,
        ],
    )
    preds16, vals16, bp16 = sc_call(class_log_probs, clp_tail, llp_pad, lp_pad)
    return (preds16[:, :BEAM], vals16[:, :BEAM], bp16[:, :BEAM])
